# baseline (speedup 1.0000x reference)
import functools

import jax
import jax.numpy as jnp
from jax import lax
from jax.experimental import pallas as pl
from jax.experimental.pallas import tpu as pltpu
from jax.experimental.pallas import tpu_sc as plsc

_L = 16
_NC = 2
_NS = 16
_NW = _NC * _NS
_RPI = 10


def _build_cnt_kernel(n_nodes, n_rows_pad, g_iters):
  acc_rows, z_per_tile, o_main, o_last = _acc_layout(n_nodes)
  mesh = plsc.VectorSubcoreMesh(core_axis_name="c", subcore_axis_name="s")

  @functools.partial(
      pl.kernel,
      mesh=mesh,
      out_type=jax.ShapeDtypeStruct((_NC, n_nodes, _L), jnp.float32),
      compiler_params=pltpu.CompilerParams(use_tc_tiling_on_sc=False),
      scratch_types=[
          pltpu.VMEM((_RPI, 128), jnp.int32),
          pltpu.VMEM((64, _L), jnp.float32),
          pltpu.VMEM((128, _L), jnp.float32),
          pltpu.VMEM_SHARED((acc_rows, _L), jnp.float32),
          pltpu.SemaphoreType.DMA,
      ],
  )
  def cnt_kernel(dst_hbm, out_hbm, dstbuf, zbuf, onesbuf, acc, sem):
    cc = lax.axis_index("c")
    ss = lax.axis_index("s")

    def fill_z(i, _):
      zbuf[i] = jnp.zeros((_L,), jnp.float32)
      return 0

    lax.fori_loop(0, 64, fill_z, 0)

    def fill_o(i, _):
      onesbuf[i] = jnp.full((_L,), 1.0, jnp.float32)
      return 0

    lax.fori_loop(0, 128, fill_o, 0)
    _zero_acc(zbuf, acc, ss, z_per_tile, sem)
    plsc.subcore_barrier()

    rows_per_tile = _RPI * g_iters
    base_row = (cc * _NS + ss) * rows_per_tile

    def edge_body(g, _):
      pltpu.sync_copy(dst_hbm.at[pl.ds(base_row + g * _RPI, _RPI)], dstbuf)
      copies = [
          pltpu.async_copy(onesbuf, acc.at[dstbuf.at[j]], sem, add=True)
          for j in range(_RPI)
      ]
      for cp in copies:
        cp.wait()
      return 0

    lax.fori_loop(0, g_iters, edge_body, 0)
    plsc.subcore_barrier()
    _copy_out(acc, out_hbm, cc, ss, o_main, o_last)

  return cnt_kernel


def _acc_layout(n_nodes):
  acc_rows = -(-(n_nodes + 1) // 128) * 128
  z_per_tile = acc_rows // _NS
  o_main = -(-n_nodes // (8 * _NS)) * 8
  o_last = n_nodes - (_NS - 1) * o_main
  return acc_rows, z_per_tile, o_main, o_last


def _zero_acc(zbuf, acc, ss, z_per_tile, sem):
  zoff = ss * z_per_tile
  z_full, z_tail = divmod(z_per_tile, 64)
  copies = []
  for c in range(z_full):
    copies.append(
        pltpu.async_copy(zbuf, acc.at[pl.ds(zoff + c * 64, 64)], sem))
    if len(copies) == 16:
      for cp in copies:
        cp.wait()
      copies = []
  if z_tail:
    copies.append(
        pltpu.async_copy(zbuf.at[pl.ds(0, z_tail)],
                         acc.at[pl.ds(zoff + z_full * 64, z_tail)], sem))
  for cp in copies:
    cp.wait()


def _copy_out(acc, out_hbm, cc, ss, o_main, o_last):
  @pl.when(ss < _NS - 1)
  def _():
    pltpu.sync_copy(acc.at[pl.ds(ss * o_main, o_main)],
                    out_hbm.at[cc, pl.ds(ss * o_main, o_main)])

  @pl.when(ss == _NS - 1)
  def _():
    pltpu.sync_copy(acc.at[pl.ds((_NS - 1) * o_main, o_last)],
                    out_hbm.at[cc, pl.ds((_NS - 1) * o_main, o_last)])


def _build_seg_kernel(n_nodes, n_rows_pad, g_iters):
  acc_rows, z_per_tile, o_main, o_last = _acc_layout(n_nodes)
  mesh = plsc.VectorSubcoreMesh(core_axis_name="c", subcore_axis_name="s")

  @functools.partial(
      pl.kernel,
      mesh=mesh,
      out_type=jax.ShapeDtypeStruct((_NC, n_nodes, _L), jnp.float32),
      compiler_params=pltpu.CompilerParams(use_tc_tiling_on_sc=False),
      scratch_types=[
          pltpu.VMEM((_RPI, 128), jnp.int32),
          pltpu.VMEM((_RPI, 128), jnp.int32),
          pltpu.VMEM((_RPI, 128, _L), jnp.float32),
          pltpu.VMEM((64, _L), jnp.float32),
          pltpu.VMEM_SHARED((acc_rows, _L), jnp.float32),
          pltpu.SemaphoreType.DMA,
          pltpu.SemaphoreType.DMA,
          pltpu.SemaphoreType.DMA,
          pltpu.SemaphoreType.DMA,
      ],
  )
  def seg_kernel(h_hbm, src_hbm, dst_hbm, out_hbm,
                 srcbuf, dstbuf, rows, zbuf, acc,
                 semg0, semg1, sems0, sems1):
    cc = lax.axis_index("c")
    ss = lax.axis_index("s")

    def zb_body(i, _):
      zbuf[i] = jnp.zeros((_L,), jnp.float32)
      return 0

    lax.fori_loop(0, 64, zb_body, 0)
    _zero_acc(zbuf, acc, ss, z_per_tile, semg0)
    plsc.subcore_barrier()

    rows_per_tile = _RPI * g_iters
    base_row = (cc * _NS + ss) * rows_per_tile
    half = _RPI // 2

    def edge_body(g, _):
      r0 = base_row + g * _RPI
      pltpu.sync_copy(src_hbm.at[pl.ds(r0, _RPI)], srcbuf)
      pltpu.sync_copy(dst_hbm.at[pl.ds(r0, _RPI)], dstbuf)
      ga = [pltpu.async_copy(h_hbm.at[srcbuf.at[j]], rows.at[j], semg0)
            for j in range(half)]
      gb = [pltpu.async_copy(h_hbm.at[srcbuf.at[j + half]], rows.at[j + half],
                             semg1)
            for j in range(half)]
      for cp in ga:
        cp.wait()
      sa = [pltpu.async_copy(rows.at[j], acc.at[dstbuf.at[j]], sems0, add=True)
            for j in range(half)]
      for cp in gb:
        cp.wait()
      sb = [pltpu.async_copy(rows.at[j + half], acc.at[dstbuf.at[j + half]],
                             sems1, add=True)
            for j in range(half)]
      for cp in sa:
        cp.wait()
      for cp in sb:
        cp.wait()
      return 0

    lax.fori_loop(0, g_iters, edge_body, 0)
    plsc.subcore_barrier()
    _copy_out(acc, out_hbm, cc, ss, o_main, o_last)

  return seg_kernel


def _row_spec(bn, d):
  return pl.BlockSpec((bn, d), lambda i: (i, 0))


def _full_spec(r, c):
  return pl.BlockSpec((r, c), lambda i: (0, 0))


def _dense_layer(p0, p1, c0, c1, h, wl_t, bl, wr_t, bn):
  n, d = h.shape

  def body(p0_r, p1_r, c0_r, c1_r, h_r, wl_r, bl_r, wr_r, o_r):
    inv = 1.0 / jnp.maximum(c0_r[...] + c1_r[...], 1.0)
    mean = (p0_r[...] + p1_r[...]) * inv
    o_r[...] = jnp.tanh(
        jnp.dot(mean, wl_r[...], preferred_element_type=jnp.float32)
        + bl_r[...]
        + jnp.dot(h_r[...], wr_r[...], preferred_element_type=jnp.float32))

  return pl.pallas_call(
      body,
      grid=(n // bn,),
      in_specs=[_row_spec(bn, d), _row_spec(bn, d),
                _row_spec(bn, 1), _row_spec(bn, 1), _row_spec(bn, d),
                _full_spec(d, d), _full_spec(1, d), _full_spec(d, d)],
      out_specs=_row_spec(bn, d),
      out_shape=jax.ShapeDtypeStruct((n, d), jnp.float32),
  )(p0, p1, c0, c1, h, wl_t, bl, wr_t)


def _head_layer(p0, p1, c0, c1, h, wl_t, bl, wr_t,
                w0_t, b0, w1_t, b1, wf_t, bf, bn):
  n, d = h.shape
  d_out = wf_t.shape[1]

  def body(p0_r, p1_r, c0_r, c1_r, h_r, wl_r, bl_r, wr_r,
           w0_r, b0_r, w1_r, b1_r, wf_r, bf_r, o_r):
    inv = 1.0 / jnp.maximum(c0_r[...] + c1_r[...], 1.0)
    mean = (p0_r[...] + p1_r[...]) * inv
    a = jnp.tanh(
        jnp.dot(mean, wl_r[...], preferred_element_type=jnp.float32)
        + bl_r[...]
        + jnp.dot(h_r[...], wr_r[...], preferred_element_type=jnp.float32))
    a = jnp.tanh(jnp.dot(a, w0_r[...], preferred_element_type=jnp.float32)
                 + b0_r[...])
    a = jnp.tanh(jnp.dot(a, w1_r[...], preferred_element_type=jnp.float32)
                 + b1_r[...])
    logits = (jnp.dot(a, wf_r[...], preferred_element_type=jnp.float32)
              + bf_r[...])
    m = jnp.max(logits, axis=1, keepdims=True)
    ex = jnp.exp(logits - m)
    o_r[...] = ex / jnp.sum(ex, axis=1, keepdims=True)

  return pl.pallas_call(
      body,
      grid=(n // bn,),
      in_specs=[_row_spec(bn, d), _row_spec(bn, d),
                _row_spec(bn, 1), _row_spec(bn, 1), _row_spec(bn, d),
                _full_spec(d, d), _full_spec(1, d), _full_spec(d, d),
                _full_spec(d, d), _full_spec(1, d),
                _full_spec(d, d), _full_spec(1, d),
                _full_spec(d, d_out), _full_spec(1, d_out)],
      out_specs=_row_spec(bn, d_out),
      out_shape=jax.ShapeDtypeStruct((n, d_out), jnp.float32),
  )(p0, p1, c0, c1, h, wl_t, bl, wr_t, w0_t, b0, w1_t, b1, wf_t, bf)


def kernel(x, edge_index, Wl0, bl0, Wr0, Wl1, bl1, Wr1, Wl2, bl2, Wr2,
           W_lin0, b_lin0, W_lin1, b_lin1, W_fin, b_fin):
  n, d = x.shape
  e = edge_index.shape[1]

  chunk_all = 128 * _RPI * _NW
  g_iters = -(-e // chunk_all)
  e_pad = g_iters * chunk_all
  src = jnp.concatenate(
      [edge_index[0], jnp.zeros((e_pad - e,), jnp.int32)]).reshape(-1, 128)
  dst = jnp.concatenate(
      [edge_index[1], jnp.full((e_pad - e,), n, jnp.int32)]).reshape(-1, 128)

  cnt_parts = _build_cnt_kernel(n, src.shape[0], g_iters)(dst)
  c0 = cnt_parts[0, :, 0:1]
  c1 = cnt_parts[1, :, 0:1]

  seg = _build_seg_kernel(n, src.shape[0], g_iters)
  bn = 4000
  layers = ((Wl0, bl0, Wr0), (Wl1, bl1, Wr1))
  h = x
  for wl, bl, wr in layers:
    parts = seg(h, src, dst)
    h = _dense_layer(parts[0], parts[1], c0, c1, h,
                     wl.T, bl.reshape(1, -1), wr.T, bn)
  parts = seg(h, src, dst)
  return _head_layer(parts[0], parts[1], c0, c1, h,
                     Wl2.T, bl2.reshape(1, -1), Wr2.T,
                     W_lin0.T, b_lin0.reshape(1, -1),
                     W_lin1.T, b_lin1.reshape(1, -1),
                     W_fin.T, b_fin.reshape(1, -1), bn)


# speedup vs baseline: 1.1612x; 1.1612x over previous
import functools

import jax
import jax.numpy as jnp
from jax import lax
from jax.experimental import pallas as pl
from jax.experimental.pallas import tpu as pltpu
from jax.experimental.pallas import tpu_sc as plsc

_L = 16
_NC = 2
_NS = 16
_NW = _NC * _NS
_RPI = 8


def _build_cnt_kernel(n_nodes, n_rows_pad, g_iters):
  acc_rows, z_per_tile, o_main, o_last = _acc_layout(n_nodes)
  mesh = plsc.VectorSubcoreMesh(core_axis_name="c", subcore_axis_name="s")

  @functools.partial(
      pl.kernel,
      mesh=mesh,
      out_type=jax.ShapeDtypeStruct((_NC, n_nodes, _L), jnp.float32),
      compiler_params=pltpu.CompilerParams(use_tc_tiling_on_sc=False),
      scratch_types=[
          pltpu.VMEM((_RPI, 128), jnp.int32),
          pltpu.VMEM((64, _L), jnp.float32),
          pltpu.VMEM((128, _L), jnp.float32),
          pltpu.VMEM_SHARED((acc_rows, _L), jnp.float32),
          pltpu.SemaphoreType.DMA,
      ],
  )
  def cnt_kernel(dst_hbm, out_hbm, dstbuf, zbuf, onesbuf, acc, sem):
    cc = lax.axis_index("c")
    ss = lax.axis_index("s")

    def fill_z(i, _):
      zbuf[i] = jnp.zeros((_L,), jnp.float32)
      return 0

    lax.fori_loop(0, 64, fill_z, 0)

    def fill_o(i, _):
      onesbuf[i] = jnp.full((_L,), 1.0, jnp.float32)
      return 0

    lax.fori_loop(0, 128, fill_o, 0)
    _zero_acc(zbuf, acc, ss, z_per_tile, sem)
    plsc.subcore_barrier()

    rows_per_tile = _RPI * g_iters
    base_row = (cc * _NS + ss) * rows_per_tile

    def edge_body(g, _):
      pltpu.sync_copy(dst_hbm.at[pl.ds(base_row + g * _RPI, _RPI)], dstbuf)
      copies = [
          pltpu.async_copy(onesbuf, acc.at[dstbuf.at[j]], sem, add=True)
          for j in range(_RPI)
      ]
      for cp in copies:
        cp.wait()
      return 0

    lax.fori_loop(0, g_iters, edge_body, 0)
    plsc.subcore_barrier()
    _copy_out(acc, out_hbm, cc, ss, o_main, o_last)

  return cnt_kernel


def _acc_layout(n_nodes):
  acc_rows = -(-(n_nodes + 1) // 128) * 128
  z_per_tile = acc_rows // _NS
  o_main = -(-n_nodes // (8 * _NS)) * 8
  o_last = n_nodes - (_NS - 1) * o_main
  return acc_rows, z_per_tile, o_main, o_last


def _zero_acc(zbuf, acc, ss, z_per_tile, sem):
  zoff = ss * z_per_tile
  z_full, z_tail = divmod(z_per_tile, 64)
  copies = []
  for c in range(z_full):
    copies.append(
        pltpu.async_copy(zbuf, acc.at[pl.ds(zoff + c * 64, 64)], sem))
    if len(copies) == 16:
      for cp in copies:
        cp.wait()
      copies = []
  if z_tail:
    copies.append(
        pltpu.async_copy(zbuf.at[pl.ds(0, z_tail)],
                         acc.at[pl.ds(zoff + z_full * 64, z_tail)], sem))
  for cp in copies:
    cp.wait()


def _copy_out(acc, out_hbm, cc, ss, o_main, o_last):
  @pl.when(ss < _NS - 1)
  def _():
    pltpu.sync_copy(acc.at[pl.ds(ss * o_main, o_main)],
                    out_hbm.at[cc, pl.ds(ss * o_main, o_main)])

  @pl.when(ss == _NS - 1)
  def _():
    pltpu.sync_copy(acc.at[pl.ds((_NS - 1) * o_main, o_last)],
                    out_hbm.at[cc, pl.ds((_NS - 1) * o_main, o_last)])


def _build_seg_kernel(n_nodes, n_rows_pad, g_iters):
  acc_rows, z_per_tile, o_main, o_last = _acc_layout(n_nodes)
  mesh = plsc.VectorSubcoreMesh(core_axis_name="c", subcore_axis_name="s")

  @functools.partial(
      pl.kernel,
      mesh=mesh,
      out_type=jax.ShapeDtypeStruct((_NC, n_nodes, _L), jnp.float32),
      compiler_params=pltpu.CompilerParams(use_tc_tiling_on_sc=False),
      scratch_types=[
          pltpu.VMEM((_RPI, 128), jnp.int32),
          pltpu.VMEM((_RPI, 128), jnp.int32),
          pltpu.VMEM((_RPI, 128, _L), jnp.float32),
          pltpu.VMEM((64, _L), jnp.float32),
          pltpu.VMEM_SHARED((acc_rows, _L), jnp.float32),
          pltpu.SemaphoreType.DMA,
          pltpu.SemaphoreType.DMA,
          pltpu.SemaphoreType.DMA,
          pltpu.SemaphoreType.DMA,
      ],
  )
  def seg_kernel(h_hbm, src_hbm, dst_hbm, out_hbm,
                 srcbuf, dstbuf, rows, zbuf, acc,
                 semg0, semg1, sems0, sems1):
    cc = lax.axis_index("c")
    ss = lax.axis_index("s")

    def zb_body(i, _):
      zbuf[i] = jnp.zeros((_L,), jnp.float32)
      return 0

    lax.fori_loop(0, 64, zb_body, 0)
    _zero_acc(zbuf, acc, ss, z_per_tile, semg0)
    plsc.subcore_barrier()

    rows_per_tile = _RPI * g_iters
    base_row = (cc * _NS + ss) * rows_per_tile
    half = _RPI // 2

    def edge_body(g, _):
      r0 = base_row + g * _RPI
      pltpu.sync_copy(src_hbm.at[pl.ds(r0, _RPI)], srcbuf)
      pltpu.sync_copy(dst_hbm.at[pl.ds(r0, _RPI)], dstbuf)
      ga = [pltpu.async_copy(h_hbm.at[srcbuf.at[j]], rows.at[j], semg0)
            for j in range(half)]
      gb = [pltpu.async_copy(h_hbm.at[srcbuf.at[j + half]], rows.at[j + half],
                             semg1)
            for j in range(half)]
      for cp in ga:
        cp.wait()
      sa = [pltpu.async_copy(rows.at[j], acc.at[dstbuf.at[j]], sems0, add=True)
            for j in range(half)]
      for cp in gb:
        cp.wait()
      sb = [pltpu.async_copy(rows.at[j + half], acc.at[dstbuf.at[j + half]],
                             sems1, add=True)
            for j in range(half)]
      for cp in sa:
        cp.wait()
      for cp in sb:
        cp.wait()
      return 0

    lax.fori_loop(0, g_iters, edge_body, 0)
    plsc.subcore_barrier()
    _copy_out(acc, out_hbm, cc, ss, o_main, o_last)

  return seg_kernel


def _row_spec(bn, d):
  return pl.BlockSpec((bn, d), lambda i: (i, 0))


def _full_spec(r, c):
  return pl.BlockSpec((r, c), lambda i: (0, 0))


def _dense_layer(p0, p1, c0, c1, h, wl_t, bl, wr_t, bn):
  n, d = h.shape

  def body(p0_r, p1_r, c0_r, c1_r, h_r, wl_r, bl_r, wr_r, o_r):
    inv = 1.0 / jnp.maximum(c0_r[...] + c1_r[...], 1.0)
    mean = (p0_r[...] + p1_r[...]) * inv
    o_r[...] = jnp.tanh(
        jnp.dot(mean, wl_r[...], preferred_element_type=jnp.float32)
        + bl_r[...]
        + jnp.dot(h_r[...], wr_r[...], preferred_element_type=jnp.float32))

  return pl.pallas_call(
      body,
      grid=(n // bn,),
      in_specs=[_row_spec(bn, d), _row_spec(bn, d),
                _row_spec(bn, 1), _row_spec(bn, 1), _row_spec(bn, d),
                _full_spec(d, d), _full_spec(1, d), _full_spec(d, d)],
      out_specs=_row_spec(bn, d),
      out_shape=jax.ShapeDtypeStruct((n, d), jnp.float32),
  )(p0, p1, c0, c1, h, wl_t, bl, wr_t)


def _head_layer(p0, p1, c0, c1, h, wl_t, bl, wr_t,
                w0_t, b0, w1_t, b1, wf_t, bf, bn):
  n, d = h.shape
  d_out = wf_t.shape[1]

  def body(p0_r, p1_r, c0_r, c1_r, h_r, wl_r, bl_r, wr_r,
           w0_r, b0_r, w1_r, b1_r, wf_r, bf_r, o_r):
    inv = 1.0 / jnp.maximum(c0_r[...] + c1_r[...], 1.0)
    mean = (p0_r[...] + p1_r[...]) * inv
    a = jnp.tanh(
        jnp.dot(mean, wl_r[...], preferred_element_type=jnp.float32)
        + bl_r[...]
        + jnp.dot(h_r[...], wr_r[...], preferred_element_type=jnp.float32))
    a = jnp.tanh(jnp.dot(a, w0_r[...], preferred_element_type=jnp.float32)
                 + b0_r[...])
    a = jnp.tanh(jnp.dot(a, w1_r[...], preferred_element_type=jnp.float32)
                 + b1_r[...])
    logits = (jnp.dot(a, wf_r[...], preferred_element_type=jnp.float32)
              + bf_r[...])
    m = jnp.max(logits, axis=1, keepdims=True)
    ex = jnp.exp(logits - m)
    o_r[...] = ex / jnp.sum(ex, axis=1, keepdims=True)

  return pl.pallas_call(
      body,
      grid=(n // bn,),
      in_specs=[_row_spec(bn, d), _row_spec(bn, d),
                _row_spec(bn, 1), _row_spec(bn, 1), _row_spec(bn, d),
                _full_spec(d, d), _full_spec(1, d), _full_spec(d, d),
                _full_spec(d, d), _full_spec(1, d),
                _full_spec(d, d), _full_spec(1, d),
                _full_spec(d, d_out), _full_spec(1, d_out)],
      out_specs=_row_spec(bn, d_out),
      out_shape=jax.ShapeDtypeStruct((n, d_out), jnp.float32),
  )(p0, p1, c0, c1, h, wl_t, bl, wr_t, w0_t, b0, w1_t, b1, wf_t, bf)


def kernel(x, edge_index, Wl0, bl0, Wr0, Wl1, bl1, Wr1, Wl2, bl2, Wr2,
           W_lin0, b_lin0, W_lin1, b_lin1, W_fin, b_fin):
  n, d = x.shape
  e = edge_index.shape[1]

  chunk_all = 128 * _RPI * _NW
  g_iters = -(-e // chunk_all)
  e_pad = g_iters * chunk_all
  src = jnp.concatenate(
      [edge_index[0], jnp.zeros((e_pad - e,), jnp.int32)]).reshape(-1, 128)
  dst = jnp.concatenate(
      [edge_index[1], jnp.full((e_pad - e,), n, jnp.int32)]).reshape(-1, 128)

  cnt_parts = _build_cnt_kernel(n, src.shape[0], g_iters)(dst)
  c0 = cnt_parts[0, :, 0:1]
  c1 = cnt_parts[1, :, 0:1]

  seg = _build_seg_kernel(n, src.shape[0], g_iters)
  bn = 4000
  layers = ((Wl0, bl0, Wr0), (Wl1, bl1, Wr1))
  h = x
  for wl, bl, wr in layers:
    parts = seg(h, src, dst)
    h = _dense_layer(parts[0], parts[1], c0, c1, h,
                     wl.T, bl.reshape(1, -1), wr.T, bn)
  parts = seg(h, src, dst)
  return _head_layer(parts[0], parts[1], c0, c1, h,
                     Wl2.T, bl2.reshape(1, -1), Wr2.T,
                     W_lin0.T, b_lin0.reshape(1, -1),
                     W_lin1.T, b_lin1.reshape(1, -1),
                     W_fin.T, b_fin.reshape(1, -1), bn)


# speedup vs baseline: 1.7294x; 1.4894x over previous
import functools

import jax
import jax.numpy as jnp
from jax import lax
from jax.experimental import pallas as pl
from jax.experimental.pallas import tpu as pltpu
from jax.experimental.pallas import tpu_sc as plsc

_L = 16
_NC = 2
_NS = 16
_NW = _NC * _NS
_RPI = 8


def _build_cnt_kernel(n_nodes, n_rows_pad, g_iters):
  acc_rows, z_per_tile, o_main, o_last = _acc_layout(n_nodes)
  mesh = plsc.VectorSubcoreMesh(core_axis_name="c", subcore_axis_name="s")

  @functools.partial(
      pl.kernel,
      mesh=mesh,
      out_type=jax.ShapeDtypeStruct((_NC, n_nodes, _L), jnp.float32),
      compiler_params=pltpu.CompilerParams(use_tc_tiling_on_sc=False),
      scratch_types=[
          pltpu.VMEM((_RPI, 128), jnp.int32),
          pltpu.VMEM((64, _L), jnp.float32),
          pltpu.VMEM((128, _L), jnp.float32),
          pltpu.VMEM_SHARED((acc_rows, _L), jnp.float32),
          pltpu.SemaphoreType.DMA,
      ],
  )
  def cnt_kernel(dst_hbm, out_hbm, dstbuf, zbuf, onesbuf, acc, sem):
    cc = lax.axis_index("c")
    ss = lax.axis_index("s")

    def fill_z(i, _):
      zbuf[i] = jnp.zeros((_L,), jnp.float32)
      return 0

    lax.fori_loop(0, 64, fill_z, 0)

    def fill_o(i, _):
      onesbuf[i] = jnp.full((_L,), 1.0, jnp.float32)
      return 0

    lax.fori_loop(0, 128, fill_o, 0)
    _zero_acc(zbuf, acc, ss, z_per_tile, sem)
    plsc.subcore_barrier()

    rows_per_tile = _RPI * g_iters
    base_row = (cc * _NS + ss) * rows_per_tile

    def edge_body(g, _):
      pltpu.sync_copy(dst_hbm.at[pl.ds(base_row + g * _RPI, _RPI)], dstbuf)
      copies = [
          pltpu.async_copy(onesbuf, acc.at[dstbuf.at[j]], sem, add=True)
          for j in range(_RPI)
      ]
      for cp in copies:
        cp.wait()
      return 0

    lax.fori_loop(0, g_iters, edge_body, 0)
    plsc.subcore_barrier()
    _copy_out(acc, out_hbm, cc, ss, o_main, o_last)

  return cnt_kernel


def _acc_layout(n_nodes):
  acc_rows = -(-(n_nodes + 1) // 128) * 128
  z_per_tile = acc_rows // _NS
  o_main = -(-n_nodes // (8 * _NS)) * 8
  o_last = n_nodes - (_NS - 1) * o_main
  return acc_rows, z_per_tile, o_main, o_last


def _zero_acc(zbuf, acc, ss, z_per_tile, sem):
  zoff = ss * z_per_tile
  z_full, z_tail = divmod(z_per_tile, 64)
  copies = []
  for c in range(z_full):
    copies.append(
        pltpu.async_copy(zbuf, acc.at[pl.ds(zoff + c * 64, 64)], sem))
    if len(copies) == 16:
      for cp in copies:
        cp.wait()
      copies = []
  if z_tail:
    copies.append(
        pltpu.async_copy(zbuf.at[pl.ds(0, z_tail)],
                         acc.at[pl.ds(zoff + z_full * 64, z_tail)], sem))
  for cp in copies:
    cp.wait()


def _copy_out(acc, out_hbm, cc, ss, o_main, o_last):
  @pl.when(ss < _NS - 1)
  def _():
    pltpu.sync_copy(acc.at[pl.ds(ss * o_main, o_main)],
                    out_hbm.at[cc, pl.ds(ss * o_main, o_main)])

  @pl.when(ss == _NS - 1)
  def _():
    pltpu.sync_copy(acc.at[pl.ds((_NS - 1) * o_main, o_last)],
                    out_hbm.at[cc, pl.ds((_NS - 1) * o_main, o_last)])


def _build_seg_kernel(n_nodes, n_rows_pad, g_iters):
  acc_rows, z_per_tile, o_main, o_last = _acc_layout(n_nodes)
  mesh = plsc.VectorSubcoreMesh(core_axis_name="c", subcore_axis_name="s")

  @functools.partial(
      pl.kernel,
      mesh=mesh,
      out_type=jax.ShapeDtypeStruct((_NC, n_nodes, _L), jnp.float32),
      compiler_params=pltpu.CompilerParams(use_tc_tiling_on_sc=False),
      scratch_types=[
          pltpu.VMEM((_RPI, 128), jnp.int32),
          pltpu.VMEM((_RPI, 128), jnp.int32),
          pltpu.VMEM((_RPI, 128, _L), jnp.float32),
          pltpu.VMEM((64, _L), jnp.float32),
          pltpu.VMEM_SHARED((acc_rows, _L), jnp.float32),
          pltpu.SemaphoreType.DMA,
          pltpu.SemaphoreType.DMA,
          pltpu.SemaphoreType.DMA,
          pltpu.SemaphoreType.DMA,
      ],
  )
  def seg_kernel(h_hbm, src_hbm, dst_hbm, out_hbm,
                 srcbuf, dstbuf, rows, zbuf, acc,
                 semg0, semg1, sems0, sems1):
    cc = lax.axis_index("c")
    ss = lax.axis_index("s")

    def zb_body(i, _):
      zbuf[i] = jnp.zeros((_L,), jnp.float32)
      return 0

    lax.fori_loop(0, 64, zb_body, 0)
    _zero_acc(zbuf, acc, ss, z_per_tile, semg0)
    plsc.subcore_barrier()

    rows_per_tile = _RPI * g_iters
    base_row = (cc * _NS + ss) * rows_per_tile
    half = _RPI // 2

    def edge_body(g, _):
      r0 = base_row + g * _RPI
      pltpu.sync_copy(src_hbm.at[pl.ds(r0, _RPI)], srcbuf)
      pltpu.sync_copy(dst_hbm.at[pl.ds(r0, _RPI)], dstbuf)
      ga = [pltpu.async_copy(h_hbm.at[srcbuf.at[j]], rows.at[j], semg0)
            for j in range(half)]
      gb = [pltpu.async_copy(h_hbm.at[srcbuf.at[j + half]], rows.at[j + half],
                             semg1)
            for j in range(half)]
      for cp in ga:
        cp.wait()
      sa = [pltpu.async_copy(rows.at[j], acc.at[dstbuf.at[j]], sems0, add=True)
            for j in range(half)]
      for cp in gb:
        cp.wait()
      sb = [pltpu.async_copy(rows.at[j + half], acc.at[dstbuf.at[j + half]],
                             sems1, add=True)
            for j in range(half)]
      for cp in sa:
        cp.wait()
      for cp in sb:
        cp.wait()
      return 0

    lax.fori_loop(0, g_iters, edge_body, 0)
    plsc.subcore_barrier()
    _copy_out(acc, out_hbm, cc, ss, o_main, o_last)

  return seg_kernel


def _dense_layer(parts, cnts, h, wlb, blb, wrb):
  nr = h.shape[0]

  def body(p_r, c_r, h_r, wl_r, bl_r, wr_r, o_r):
    inv = 1.0 / jnp.maximum(c_r[0] + c_r[1], 1.0)
    mean = (p_r[0] + p_r[1]) * inv
    o_r[...] = jnp.tanh(
        jnp.dot(mean, wl_r[...], preferred_element_type=jnp.float32)
        + bl_r[...]
        + jnp.dot(h_r[...], wr_r[...], preferred_element_type=jnp.float32))

  return pl.pallas_call(
      body,
      out_shape=jax.ShapeDtypeStruct((nr, 128), jnp.float32),
  )(parts, cnts, h, wlb, blb, wrb)


def _head_layer(parts, cnts, h, wlb, blb, wrb, w0b, b0b, w1b, b1b, wfb, bfb,
                pair_sum):
  nr = h.shape[0]

  def body(p_r, c_r, h_r, wl_r, bl_r, wr_r, w0_r, b0_r, w1_r, b1_r,
           wf_r, bf_r, ps_r, o_r):
    inv = 1.0 / jnp.maximum(c_r[0] + c_r[1], 1.0)
    mean = (p_r[0] + p_r[1]) * inv
    a = jnp.tanh(
        jnp.dot(mean, wl_r[...], preferred_element_type=jnp.float32)
        + bl_r[...]
        + jnp.dot(h_r[...], wr_r[...], preferred_element_type=jnp.float32))
    a = jnp.tanh(jnp.dot(a, w0_r[...], preferred_element_type=jnp.float32)
                 + b0_r[...])
    a = jnp.tanh(jnp.dot(a, w1_r[...], preferred_element_type=jnp.float32)
                 + b1_r[...])
    logits = (jnp.dot(a, wf_r[...], preferred_element_type=jnp.float32)
              + bf_r[...])
    ex = jnp.exp(logits)
    denom = jnp.dot(ex, ps_r[...], preferred_element_type=jnp.float32)
    o_r[...] = ex / denom

  return pl.pallas_call(
      body,
      out_shape=jax.ShapeDtypeStruct((nr, 16), jnp.float32),
  )(parts, cnts, h, wlb, blb, wrb, w0b, b0b, w1b, b1b, wfb, bfb, pair_sum)


def kernel(x, edge_index, Wl0, bl0, Wr0, Wl1, bl1, Wr1, Wl2, bl2, Wr2,
           W_lin0, b_lin0, W_lin1, b_lin1, W_fin, b_fin):
  n, d = x.shape
  e = edge_index.shape[1]
  npk = 128 // d
  nr = n // npk

  chunk_all = 128 * _RPI * _NW
  g_iters = -(-e // chunk_all)
  e_pad = g_iters * chunk_all
  src = jnp.concatenate(
      [edge_index[0], jnp.zeros((e_pad - e,), jnp.int32)]).reshape(-1, 128)
  dst = jnp.concatenate(
      [edge_index[1], jnp.full((e_pad - e,), n, jnp.int32)]).reshape(-1, 128)

  cnt_parts = _build_cnt_kernel(n, src.shape[0], g_iters)(dst)
  cnts = cnt_parts.reshape(_NC, nr, 128)

  seg = _build_seg_kernel(n, src.shape[0], g_iters)
  eye = jnp.eye(npk, dtype=jnp.float32)

  def blkdiag(w):
    return jnp.kron(eye, w.T)

  def blkbias(b):
    return jnp.tile(b, npk).reshape(1, -1)

  h = x
  for wl, bl, wr in ((Wl0, bl0, Wr0), (Wl1, bl1, Wr1)):
    parts = seg(h, src, dst).reshape(_NC, nr, 128)
    h = _dense_layer(parts, cnts, h.reshape(nr, 128),
                     blkdiag(wl), blkbias(bl), blkdiag(wr)).reshape(n, d)
  parts = seg(h, src, dst).reshape(_NC, nr, 128)
  d_out = W_fin.shape[0]
  pair_sum = jnp.kron(eye, jnp.ones((d_out, d_out), jnp.float32))
  out = _head_layer(parts, cnts, h.reshape(nr, 128),
                    blkdiag(Wl2), blkbias(bl2), blkdiag(Wr2),
                    blkdiag(W_lin0), blkbias(b_lin0),
                    blkdiag(W_lin1), blkbias(b_lin1),
                    jnp.kron(eye, W_fin.T), blkbias(b_fin), pair_sum)
  return out.reshape(n, d_out)


# speedup vs baseline: 1.9574x; 1.1318x over previous
import functools

import jax
import jax.numpy as jnp
from jax import lax
from jax.experimental import pallas as pl
from jax.experimental.pallas import tpu as pltpu
from jax.experimental.pallas import tpu_sc as plsc

_L = 16
_NC = 2
_NS = 16
_NW = _NC * _NS
_RPI = 8


def _build_cnt_kernel(n_nodes, n_rows, n_full, n_extra):
  acc_rows, z_per_tile, o_main, o_last = _acc_layout(n_nodes)
  mesh = plsc.VectorSubcoreMesh(core_axis_name="c", subcore_axis_name="s")

  @functools.partial(
      pl.kernel,
      mesh=mesh,
      out_type=jax.ShapeDtypeStruct((_NC, n_nodes, _L), jnp.float32),
      compiler_params=pltpu.CompilerParams(use_tc_tiling_on_sc=False),
      scratch_types=[
          pltpu.VMEM((_RPI, 128), jnp.int32),
          pltpu.VMEM((64, _L), jnp.float32),
          pltpu.VMEM((128, _L), jnp.float32),
          pltpu.VMEM_SHARED((acc_rows, _L), jnp.float32),
          pltpu.SemaphoreType.DMA,
      ],
  )
  def cnt_kernel(dst_hbm, out_hbm, dstbuf, zbuf, onesbuf, acc, sem):
    cc = lax.axis_index("c")
    ss = lax.axis_index("s")

    def fill_z(i, _):
      zbuf[i] = jnp.zeros((_L,), jnp.float32)
      return 0

    lax.fori_loop(0, 64, fill_z, 0)

    def fill_o(i, _):
      onesbuf[i] = jnp.full((_L,), 1.0, jnp.float32)
      return 0

    lax.fori_loop(0, 128, fill_o, 0)
    _zero_acc(zbuf, acc, ss, z_per_tile, sem)
    plsc.subcore_barrier()

    wid = cc * _NS + ss
    base_row = wid * (n_full * _RPI)

    def process(r0):
      pltpu.sync_copy(dst_hbm.at[pl.ds(r0, _RPI)], dstbuf)
      copies = [
          pltpu.async_copy(onesbuf, acc.at[dstbuf.at[j]], sem, add=True)
          for j in range(_RPI)
      ]
      for cp in copies:
        cp.wait()

    def edge_body(g, _):
      process(base_row + g * _RPI)
      return 0

    lax.fori_loop(0, n_full, edge_body, 0)
    if n_extra:
      @pl.when(wid < n_extra)
      def _():
        process(_NW * n_full * _RPI + wid * _RPI)
    plsc.subcore_barrier()
    _copy_out(acc, out_hbm, cc, ss, o_main, o_last)

  return cnt_kernel


def _acc_layout(n_nodes):
  acc_rows = -(-(n_nodes + 1) // 128) * 128
  z_per_tile = acc_rows // _NS
  o_main = -(-n_nodes // (8 * _NS)) * 8
  o_last = n_nodes - (_NS - 1) * o_main
  return acc_rows, z_per_tile, o_main, o_last


def _zero_acc(zbuf, acc, ss, z_per_tile, sem):
  zoff = ss * z_per_tile
  z_full, z_tail = divmod(z_per_tile, 64)
  copies = []
  for c in range(z_full):
    copies.append(
        pltpu.async_copy(zbuf, acc.at[pl.ds(zoff + c * 64, 64)], sem))
    if len(copies) == 16:
      for cp in copies:
        cp.wait()
      copies = []
  if z_tail:
    copies.append(
        pltpu.async_copy(zbuf.at[pl.ds(0, z_tail)],
                         acc.at[pl.ds(zoff + z_full * 64, z_tail)], sem))
  for cp in copies:
    cp.wait()


def _copy_out(acc, out_hbm, cc, ss, o_main, o_last):
  @pl.when(ss < _NS - 1)
  def _():
    pltpu.sync_copy(acc.at[pl.ds(ss * o_main, o_main)],
                    out_hbm.at[cc, pl.ds(ss * o_main, o_main)])

  @pl.when(ss == _NS - 1)
  def _():
    pltpu.sync_copy(acc.at[pl.ds((_NS - 1) * o_main, o_last)],
                    out_hbm.at[cc, pl.ds((_NS - 1) * o_main, o_last)])


def _build_seg_kernel(n_nodes, n_rows, n_full, n_extra):
  acc_rows, z_per_tile, o_main, o_last = _acc_layout(n_nodes)
  mesh = plsc.VectorSubcoreMesh(core_axis_name="c", subcore_axis_name="s")

  @functools.partial(
      pl.kernel,
      mesh=mesh,
      out_type=jax.ShapeDtypeStruct((_NC, n_nodes, _L), jnp.float32),
      compiler_params=pltpu.CompilerParams(use_tc_tiling_on_sc=False),
      scratch_types=[
          pltpu.VMEM((_RPI, 128), jnp.int32),
          pltpu.VMEM((_RPI, 128), jnp.int32),
          pltpu.VMEM((_RPI, 128, _L), jnp.float32),
          pltpu.VMEM((64, _L), jnp.float32),
          pltpu.VMEM_SHARED((acc_rows, _L), jnp.float32),
          pltpu.SemaphoreType.DMA,
          pltpu.SemaphoreType.DMA,
          pltpu.SemaphoreType.DMA,
          pltpu.SemaphoreType.DMA,
      ],
  )
  def seg_kernel(h_hbm, src_hbm, dst_hbm, out_hbm,
                 srcbuf, dstbuf, rows, zbuf, acc,
                 semg0, semg1, sems0, sems1):
    cc = lax.axis_index("c")
    ss = lax.axis_index("s")

    def zb_body(i, _):
      zbuf[i] = jnp.zeros((_L,), jnp.float32)
      return 0

    lax.fori_loop(0, 64, zb_body, 0)
    _zero_acc(zbuf, acc, ss, z_per_tile, semg0)
    plsc.subcore_barrier()

    wid = cc * _NS + ss
    base_row = wid * (n_full * _RPI)
    half = _RPI // 2

    def process(r0):
      pltpu.sync_copy(src_hbm.at[pl.ds(r0, _RPI)], srcbuf)
      pltpu.sync_copy(dst_hbm.at[pl.ds(r0, _RPI)], dstbuf)
      ga = [pltpu.async_copy(h_hbm.at[srcbuf.at[j]], rows.at[j], semg0)
            for j in range(half)]
      gb = [pltpu.async_copy(h_hbm.at[srcbuf.at[j + half]], rows.at[j + half],
                             semg1)
            for j in range(half)]
      for cp in ga:
        cp.wait()
      sa = [pltpu.async_copy(rows.at[j], acc.at[dstbuf.at[j]], sems0, add=True)
            for j in range(half)]
      for cp in gb:
        cp.wait()
      sb = [pltpu.async_copy(rows.at[j + half], acc.at[dstbuf.at[j + half]],
                             sems1, add=True)
            for j in range(half)]
      for cp in sa:
        cp.wait()
      for cp in sb:
        cp.wait()

    def edge_body(g, _):
      process(base_row + g * _RPI)
      return 0

    lax.fori_loop(0, n_full, edge_body, 0)
    if n_extra:
      @pl.when(wid < n_extra)
      def _():
        process(_NW * n_full * _RPI + wid * _RPI)
    plsc.subcore_barrier()
    _copy_out(acc, out_hbm, cc, ss, o_main, o_last)

  return seg_kernel


def _dense_layer(parts, cnts, h, wlb, blb, wrb):
  nr = h.shape[0]

  def body(p_r, c_r, h_r, wl_r, bl_r, wr_r, o_r):
    inv = 1.0 / jnp.maximum(c_r[0] + c_r[1], 1.0)
    mean = (p_r[0] + p_r[1]) * inv
    o_r[...] = jnp.tanh(
        jnp.dot(mean, wl_r[...], preferred_element_type=jnp.float32)
        + bl_r[...]
        + jnp.dot(h_r[...], wr_r[...], preferred_element_type=jnp.float32))

  return pl.pallas_call(
      body,
      out_shape=jax.ShapeDtypeStruct((nr, 128), jnp.float32),
  )(parts, cnts, h, wlb, blb, wrb)


def _head_layer(parts, cnts, h, wlb, blb, wrb, w0b, b0b, w1b, b1b, wfb, bfb,
                pair_sum):
  nr = h.shape[0]

  def body(p_r, c_r, h_r, wl_r, bl_r, wr_r, w0_r, b0_r, w1_r, b1_r,
           wf_r, bf_r, ps_r, o_r):
    inv = 1.0 / jnp.maximum(c_r[0] + c_r[1], 1.0)
    mean = (p_r[0] + p_r[1]) * inv
    a = jnp.tanh(
        jnp.dot(mean, wl_r[...], preferred_element_type=jnp.float32)
        + bl_r[...]
        + jnp.dot(h_r[...], wr_r[...], preferred_element_type=jnp.float32))
    a = jnp.tanh(jnp.dot(a, w0_r[...], preferred_element_type=jnp.float32)
                 + b0_r[...])
    a = jnp.tanh(jnp.dot(a, w1_r[...], preferred_element_type=jnp.float32)
                 + b1_r[...])
    logits = (jnp.dot(a, wf_r[...], preferred_element_type=jnp.float32)
              + bf_r[...])
    ex = jnp.exp(logits)
    denom = jnp.dot(ex, ps_r[...], preferred_element_type=jnp.float32)
    o_r[...] = ex / denom

  return pl.pallas_call(
      body,
      out_shape=jax.ShapeDtypeStruct((nr, 16), jnp.float32),
  )(parts, cnts, h, wlb, blb, wrb, w0b, b0b, w1b, b1b, wfb, bfb, pair_sum)


def kernel(x, edge_index, Wl0, bl0, Wr0, Wl1, bl1, Wr1, Wl2, bl2, Wr2,
           W_lin0, b_lin0, W_lin1, b_lin1, W_fin, b_fin):
  n, d = x.shape
  e = edge_index.shape[1]
  npk = 128 // d
  nr = n // npk

  if e % (128 * _RPI) == 0:
    src = edge_index[0].reshape(-1, 128)
    dst = edge_index[1].reshape(-1, 128)
  else:
    chunk_all = 128 * _RPI * _NW
    e_pad = -(-e // chunk_all) * chunk_all
    src = jnp.concatenate(
        [edge_index[0], jnp.zeros((e_pad - e,), jnp.int32)]).reshape(-1, 128)
    dst = jnp.concatenate(
        [edge_index[1], jnp.full((e_pad - e,), n, jnp.int32)]).reshape(-1, 128)
  n_rows = src.shape[0]
  n_full = n_rows // (_NW * _RPI)
  n_extra = (n_rows - n_full * _NW * _RPI) // _RPI

  cnt_parts = _build_cnt_kernel(n, n_rows, n_full, n_extra)(dst)
  cnts = cnt_parts.reshape(_NC, nr, 128)

  seg = _build_seg_kernel(n, n_rows, n_full, n_extra)
  eye = jnp.eye(npk, dtype=jnp.float32)

  def blkdiag(w):
    return jnp.kron(eye, w.T)

  def blkbias(b):
    return jnp.tile(b, npk).reshape(1, -1)

  h = x
  for wl, bl, wr in ((Wl0, bl0, Wr0), (Wl1, bl1, Wr1)):
    parts = seg(h, src, dst).reshape(_NC, nr, 128)
    h = _dense_layer(parts, cnts, h.reshape(nr, 128),
                     blkdiag(wl), blkbias(bl), blkdiag(wr)).reshape(n, d)
  parts = seg(h, src, dst).reshape(_NC, nr, 128)
  d_out = W_fin.shape[0]
  pair_sum = jnp.kron(eye, jnp.ones((d_out, d_out), jnp.float32))
  out = _head_layer(parts, cnts, h.reshape(nr, 128),
                    blkdiag(Wl2), blkbias(bl2), blkdiag(Wr2),
                    blkdiag(W_lin0), blkbias(b_lin0),
                    blkdiag(W_lin1), blkbias(b_lin1),
                    jnp.kron(eye, W_fin.T), blkbias(b_fin), pair_sum)
  return out.reshape(n, d_out)
